# baseline (speedup 1.0000x reference)
import functools

import jax
import jax.numpy as jnp
from jax import lax
from jax.experimental import pallas as pl
from jax.experimental.pallas import tpu as pltpu
from jax.experimental.pallas import tpu_sc as plsc

N = 10000
NPAD = 10240
E = 320000
FH = 128
CHUNK = 128
CPS = 160
ROWS2D = 16 * CPS
SB = 40
DB = 80
DW = FH
TRASH = NPAD - 1
RPS = NPAD // 16
RB = 256
GRID = NPAD // RB

_mesh = plsc.VectorSubcoreMesh(core_axis_name="c", subcore_axis_name="s")


@functools.partial(
    pl.kernel,
    out_type=jax.ShapeDtypeStruct((2, NPAD, DW), jnp.float32),
    mesh=_mesh,
    scratch_types=[
        pltpu.VMEM_SHARED((NPAD, DW), jnp.float32),
        pltpu.VMEM((DB, CHUNK), jnp.int32),
        pltpu.VMEM((CHUNK, DW), jnp.float32),
        pltpu.SemaphoreType.DMA,
    ],
)
def _sc_deg(dst2d, ones_blk, zeros64, deg_out, dacc, didx, ones_v, sem):
    c = lax.axis_index("c")
    s = lax.axis_index("s")
    base = s * RPS
    pltpu.sync_copy(ones_blk, ones_v)

    def zfire(i, _):
        pltpu.async_copy(zeros64, dacc.at[pl.ds(base + i * 64, 64)], sem)
        return 0

    lax.fori_loop(0, RPS // 64, zfire, 0)

    def zdrain(i, _):
        pltpu.make_async_copy(zeros64, dacc.at[pl.ds(base, 64)], sem).wait()
        return 0

    lax.fori_loop(0, RPS // 64, zdrain, 0)
    plsc.subcore_barrier()

    half = CPS // 2

    def blk(j, _):
        pltpu.sync_copy(
            dst2d.at[pl.ds(s * CPS + c * half + j * DB, DB)], didx)

        def fire(k, _):
            pltpu.async_copy(ones_v, dacc.at[didx.at[k]], sem, add=True)
            return 0

        lax.fori_loop(0, DB, fire, 0)

        def drain(k, _):
            pltpu.make_async_copy(ones_v, dacc.at[didx.at[0]], sem).wait()
            return 0

        lax.fori_loop(0, DB, drain, 0)
        return 0

    lax.fori_loop(0, half // DB, blk, 0)
    plsc.subcore_barrier()
    pltpu.sync_copy(dacc.at[pl.ds(base, RPS)],
                    deg_out.at[c].at[pl.ds(base, RPS)])


@functools.partial(
    pl.kernel,
    out_type=jax.ShapeDtypeStruct((2, NPAD, FH), jnp.float32),
    mesh=_mesh,
    scratch_types=[
        pltpu.VMEM_SHARED((NPAD, FH), jnp.float32),
        pltpu.VMEM((SB, CHUNK), jnp.int32),
        pltpu.VMEM((SB, CHUNK), jnp.int32),
        pltpu.VMEM((CHUNK, FH), jnp.float32),
        pltpu.VMEM((CHUNK, FH), jnp.float32),
        pltpu.SemaphoreType.DMA,
        pltpu.SemaphoreType.DMA,
    ],
)
def _sc_agg(gst, src2d, dst2d, zeros64, sst,
            acc, sidx, didx, rows_a, rows_b, sem_a, sem_b):
    c = lax.axis_index("c")
    s = lax.axis_index("s")
    base = s * RPS
    g = gst.at[c]

    def zfire(i, _):
        pltpu.async_copy(zeros64, acc.at[pl.ds(base + i * 64, 64)], sem_a)
        return 0

    lax.fori_loop(0, RPS // 64, zfire, 0)

    def zdrain(i, _):
        pltpu.make_async_copy(zeros64, acc.at[pl.ds(base, 64)], sem_a).wait()
        return 0

    lax.fori_loop(0, RPS // 64, zdrain, 0)
    plsc.subcore_barrier()

    def blk(j, _):
        rowbase = s * CPS + j * SB
        pltpu.sync_copy(src2d.at[pl.ds(rowbase, SB)], sidx)
        pltpu.sync_copy(dst2d.at[pl.ds(rowbase, SB)], didx)
        pltpu.async_copy(g.at[sidx.at[0]], rows_a, sem_a)

        def pair(p, _):
            k0 = p * 2
            pltpu.async_copy(g.at[sidx.at[k0 + 1]], rows_b, sem_b)
            pltpu.make_async_copy(g.at[sidx.at[k0]], rows_a, sem_a).wait()
            pltpu.sync_copy(rows_a, acc.at[didx.at[k0]], add=True)

            @pl.when(p + 1 < SB // 2)
            def _():
                pltpu.async_copy(g.at[sidx.at[k0 + 2]], rows_a, sem_a)

            pltpu.make_async_copy(g.at[sidx.at[k0 + 1]], rows_b, sem_b).wait()
            pltpu.sync_copy(rows_b, acc.at[didx.at[k0 + 1]], add=True)
            return 0

        lax.fori_loop(0, SB // 2, pair, 0)
        return 0

    lax.fori_loop(0, CPS // SB, blk, 0)
    plsc.subcore_barrier()
    sl = pl.ds(base, RPS)
    pltpu.sync_copy(acc.at[sl], sst.at[c].at[sl])


def _tc1_body(x_ref, w_ref, deg_ref, g_ref, dinv_ref):
    deg = deg_ref[0][:, 0:1] + deg_ref[1][:, 0:1] + 1.0
    dinv = lax.rsqrt(deg)
    dinv_ref[...] = jnp.broadcast_to(dinv, (RB, 8))
    t = jnp.dot(x_ref[...], w_ref[...], preferred_element_type=jnp.float32)
    g = t * dinv
    g_ref[0] = g[:, :FH]
    g_ref[1] = g[:, FH:]


_tc1 = pl.pallas_call(
    _tc1_body,
    grid=(GRID,),
    in_specs=[
        pl.BlockSpec((RB, 128), lambda i: (i, 0)),
        pl.BlockSpec((128, 256), lambda i: (0, 0)),
        pl.BlockSpec((2, RB, DW), lambda i: (0, i, 0)),
    ],
    out_specs=[
        pl.BlockSpec((2, RB, FH), lambda i: (0, i, 0)),
        pl.BlockSpec((RB, 8), lambda i: (i, 0)),
    ],
    out_shape=[
        jax.ShapeDtypeStruct((2, NPAD, FH), jnp.float32),
        jax.ShapeDtypeStruct((NPAD, 8), jnp.float32),
    ],
)


def _tcmid_body(s_ref, g_ref, dinv_ref, b_ref, w_ref, o_ref):
    dinv = dinv_ref[...][:, 0:1]
    b = b_ref[...]
    hl = (s_ref[0] + g_ref[0]) * dinv + b[:, :FH]
    hr = (s_ref[1] + g_ref[1]) * dinv + b[:, FH:]
    pleft = jnp.maximum(hl, 0.0)
    pright = jnp.maximum(hr, 0.0)
    w = w_ref[...]
    t = (jnp.dot(pleft, w[:FH, :], preferred_element_type=jnp.float32)
         + jnp.dot(pright, w[FH:, :], preferred_element_type=jnp.float32))
    g = t * dinv
    o_ref[0] = g[:, :FH]
    o_ref[1] = g[:, FH:]


_tcmid = pl.pallas_call(
    _tcmid_body,
    grid=(GRID,),
    in_specs=[
        pl.BlockSpec((2, RB, FH), lambda i: (0, i, 0)),
        pl.BlockSpec((2, RB, FH), lambda i: (0, i, 0)),
        pl.BlockSpec((RB, 8), lambda i: (i, 0)),
        pl.BlockSpec((1, 256), lambda i: (0, 0)),
        pl.BlockSpec((256, 256), lambda i: (0, 0)),
    ],
    out_specs=pl.BlockSpec((2, RB, FH), lambda i: (0, i, 0)),
    out_shape=jax.ShapeDtypeStruct((2, NPAD, FH), jnp.float32),
)


def _tcfin_body(s_ref, g_ref, dinv_ref, b_ref, o_ref):
    dinv = dinv_ref[...][:, 0:1]
    b = b_ref[...]
    hl = (s_ref[0] + g_ref[0]) * dinv + b[:, :FH]
    hr = (s_ref[1] + g_ref[1]) * dinv + b[:, FH:]
    n2 = (jnp.sum(hl * hl, axis=1, keepdims=True)
          + jnp.sum(hr * hr, axis=1, keepdims=True))
    inv = 1.0 / jnp.maximum(jnp.sqrt(n2), 1e-12)
    o_ref[...] = jnp.concatenate([hl * inv, hr * inv], axis=1)


_tcfin = pl.pallas_call(
    _tcfin_body,
    grid=(GRID,),
    in_specs=[
        pl.BlockSpec((2, RB, FH), lambda i: (0, i, 0)),
        pl.BlockSpec((2, RB, FH), lambda i: (0, i, 0)),
        pl.BlockSpec((RB, 8), lambda i: (i, 0)),
        pl.BlockSpec((1, 256), lambda i: (0, 0)),
    ],
    out_specs=pl.BlockSpec((RB, 256), lambda i: (i, 0)),
    out_shape=jax.ShapeDtypeStruct((NPAD, 256), jnp.float32),
)


def kernel(x, edge_index, W1, b1, W2, b2, W3, b3, W4, b4, W5, b5, W6, b6):
    x_p = jnp.zeros((NPAD, 128), jnp.float32).at[:N].set(x)
    src = edge_index[0]
    dst = edge_index[1]
    npad_e = ROWS2D * CHUNK - E
    pidx = jnp.arange(npad_e, dtype=jnp.int32)
    src2d = jnp.concatenate(
        [src, pidx % N]).reshape(ROWS2D, CHUNK)
    dst2d = jnp.concatenate(
        [dst, N + pidx % (NPAD - N)]).reshape(ROWS2D, CHUNK)
    ones_blk = jnp.ones((CHUNK, DW), jnp.float32)
    zeros64 = jnp.zeros((64, FH), jnp.float32)
    zeros16 = jnp.zeros((64, DW), jnp.float32)

    deg = _sc_deg(dst2d, ones_blk, zeros16)
    g, dinv8 = _tc1(x_p, W1, deg)
    for wn, bn in ((W2, b1), (W3, b2), (W4, b3), (W5, b4), (W6, b5)):
        sst = _sc_agg(g, src2d, dst2d, zeros64)
        g = _tcmid(sst, g, dinv8, bn.reshape(1, 256), wn)
    sst = _sc_agg(g, src2d, dst2d, zeros64)
    out = _tcfin(sst, g, dinv8, b6.reshape(1, 256))
    return out[:N]


# speedup vs baseline: 1.0019x; 1.0019x over previous
import functools

import jax
import jax.numpy as jnp
from jax import lax
from jax.experimental import pallas as pl
from jax.experimental.pallas import tpu as pltpu
from jax.experimental.pallas import tpu_sc as plsc

N = 10000
NPAD = 10240
E = 320000
FH = 128
CHUNK = 128
CPS = 160
ROWS2D = 16 * CPS
SB = 40
DB = 80
DW = FH
RPS = NPAD // 16
RB = 256
GRID = NPAD // RB

_mesh = plsc.VectorSubcoreMesh(core_axis_name="c", subcore_axis_name="s")


@functools.partial(
    pl.kernel,
    out_type=jax.ShapeDtypeStruct((2, NPAD, DW), jnp.float32),
    mesh=_mesh,
    scratch_types=[
        pltpu.VMEM_SHARED((NPAD, DW), jnp.float32),
        pltpu.VMEM((DB, CHUNK), jnp.int32),
        pltpu.VMEM((CHUNK, DW), jnp.float32),
        pltpu.SemaphoreType.DMA,
    ],
)
def _sc_deg(dst2d, ones_blk, zeros64, deg_out, dacc, didx, ones_v, sem):
    c = lax.axis_index("c")
    s = lax.axis_index("s")
    base = s * RPS
    pltpu.sync_copy(ones_blk, ones_v)

    def zfire(i, _):
        pltpu.async_copy(zeros64, dacc.at[pl.ds(base + i * 64, 64)], sem)
        return 0

    lax.fori_loop(0, RPS // 64, zfire, 0)

    def zdrain(i, _):
        pltpu.make_async_copy(zeros64, dacc.at[pl.ds(base, 64)], sem).wait()
        return 0

    lax.fori_loop(0, RPS // 64, zdrain, 0)
    plsc.subcore_barrier()

    half = CPS // 2

    def blk(j, _):
        pltpu.sync_copy(
            dst2d.at[pl.ds(s * CPS + c * half + j * DB, DB)], didx)

        def fire(k, _):
            pltpu.async_copy(ones_v, dacc.at[didx.at[k]], sem, add=True)
            return 0

        lax.fori_loop(0, DB, fire, 0)

        def drain(k, _):
            pltpu.make_async_copy(ones_v, dacc.at[didx.at[0]], sem).wait()
            return 0

        lax.fori_loop(0, DB, drain, 0)
        return 0

    lax.fori_loop(0, half // DB, blk, 0)
    plsc.subcore_barrier()
    pltpu.sync_copy(dacc.at[pl.ds(base, RPS)],
                    deg_out.at[c].at[pl.ds(base, RPS)])


@functools.partial(
    pl.kernel,
    out_type=jax.ShapeDtypeStruct((2, NPAD, FH), jnp.float32),
    mesh=_mesh,
    scratch_types=[
        pltpu.VMEM_SHARED((NPAD, FH), jnp.float32),
        pltpu.VMEM((SB, CHUNK), jnp.int32),
        pltpu.VMEM((SB, CHUNK), jnp.int32),
        pltpu.VMEM((CHUNK, FH), jnp.float32),
        pltpu.VMEM((CHUNK, FH), jnp.float32),
        pltpu.SemaphoreType.DMA,
        pltpu.SemaphoreType.DMA,
    ],
)
def _sc_agg(gst, src2d, dst2d, zeros64, sst,
            acc, sidx, didx, rows_a, rows_b, sem_a, sem_b):
    c = lax.axis_index("c")
    s = lax.axis_index("s")
    base = s * RPS
    g = gst.at[c]

    def zfire(i, _):
        pltpu.async_copy(zeros64, acc.at[pl.ds(base + i * 64, 64)], sem_a)
        return 0

    lax.fori_loop(0, RPS // 64, zfire, 0)

    def zdrain(i, _):
        pltpu.make_async_copy(zeros64, acc.at[pl.ds(base, 64)], sem_a).wait()
        return 0

    lax.fori_loop(0, RPS // 64, zdrain, 0)
    plsc.subcore_barrier()

    def blk(j, _):
        rowbase = s * CPS + j * SB
        pltpu.sync_copy(src2d.at[pl.ds(rowbase, SB)], sidx)
        pltpu.sync_copy(dst2d.at[pl.ds(rowbase, SB)], didx)
        pltpu.async_copy(g.at[sidx.at[0]], rows_a, sem_a)

        def pair(p, _):
            k0 = p * 2
            pltpu.async_copy(g.at[sidx.at[k0 + 1]], rows_b, sem_b)
            pltpu.make_async_copy(g.at[sidx.at[k0]], rows_a, sem_a).wait()
            pltpu.sync_copy(rows_a, acc.at[didx.at[k0]], add=True)

            @pl.when(p + 1 < SB // 2)
            def _():
                pltpu.async_copy(g.at[sidx.at[k0 + 2]], rows_a, sem_a)

            pltpu.make_async_copy(g.at[sidx.at[k0 + 1]], rows_b, sem_b).wait()
            pltpu.sync_copy(rows_b, acc.at[didx.at[k0 + 1]], add=True)
            return 0

        lax.fori_loop(0, SB // 2, pair, 0)
        return 0

    lax.fori_loop(0, CPS // SB, blk, 0)
    plsc.subcore_barrier()
    sl = pl.ds(base, RPS)
    pltpu.sync_copy(acc.at[sl], sst.at[c].at[sl])


def _tc1_body(x_ref, w_ref, deg_ref, g_ref, dinv_ref):
    deg = deg_ref[0][:, 0:1] + deg_ref[1][:, 0:1] + 1.0
    dinv = lax.rsqrt(deg)
    dinv_ref[...] = jnp.broadcast_to(dinv, (RB, 8))
    t = jnp.dot(x_ref[...], w_ref[...], preferred_element_type=jnp.float32)
    g = t * dinv
    g_ref[0] = g[:, :FH]
    g_ref[1] = g[:, FH:]


_tc1 = pl.pallas_call(
    _tc1_body,
    grid=(GRID,),
    in_specs=[
        pl.BlockSpec((RB, 128), lambda i: (i, 0)),
        pl.BlockSpec((128, 256), lambda i: (0, 0)),
        pl.BlockSpec((2, RB, DW), lambda i: (0, i, 0)),
    ],
    out_specs=[
        pl.BlockSpec((2, RB, FH), lambda i: (0, i, 0)),
        pl.BlockSpec((RB, 8), lambda i: (i, 0)),
    ],
    out_shape=[
        jax.ShapeDtypeStruct((2, NPAD, FH), jnp.float32),
        jax.ShapeDtypeStruct((NPAD, 8), jnp.float32),
    ],
)


def _tcmid_body(s_ref, g_ref, dinv_ref, b_ref, w_ref, o_ref):
    dinv = dinv_ref[...][:, 0:1]
    b = b_ref[...]
    hl = (s_ref[0] + g_ref[0]) * dinv + b[:, :FH]
    hr = (s_ref[1] + g_ref[1]) * dinv + b[:, FH:]
    pleft = jnp.maximum(hl, 0.0)
    pright = jnp.maximum(hr, 0.0)
    w = w_ref[...]
    t = (jnp.dot(pleft, w[:FH, :], preferred_element_type=jnp.float32)
         + jnp.dot(pright, w[FH:, :], preferred_element_type=jnp.float32))
    g = t * dinv
    o_ref[0] = g[:, :FH]
    o_ref[1] = g[:, FH:]


_tcmid = pl.pallas_call(
    _tcmid_body,
    grid=(GRID,),
    in_specs=[
        pl.BlockSpec((2, RB, FH), lambda i: (0, i, 0)),
        pl.BlockSpec((2, RB, FH), lambda i: (0, i, 0)),
        pl.BlockSpec((RB, 8), lambda i: (i, 0)),
        pl.BlockSpec((1, 256), lambda i: (0, 0)),
        pl.BlockSpec((256, 256), lambda i: (0, 0)),
    ],
    out_specs=pl.BlockSpec((2, RB, FH), lambda i: (0, i, 0)),
    out_shape=jax.ShapeDtypeStruct((2, NPAD, FH), jnp.float32),
)


def _tcfin_body(s_ref, g_ref, dinv_ref, b_ref, o_ref):
    dinv = dinv_ref[...][:, 0:1]
    b = b_ref[...]
    hl = (s_ref[0] + g_ref[0]) * dinv + b[:, :FH]
    hr = (s_ref[1] + g_ref[1]) * dinv + b[:, FH:]
    n2 = (jnp.sum(hl * hl, axis=1, keepdims=True)
          + jnp.sum(hr * hr, axis=1, keepdims=True))
    inv = 1.0 / jnp.maximum(jnp.sqrt(n2), 1e-12)
    o_ref[...] = jnp.concatenate([hl * inv, hr * inv], axis=1)


_tcfin = pl.pallas_call(
    _tcfin_body,
    grid=(GRID,),
    in_specs=[
        pl.BlockSpec((2, RB, FH), lambda i: (0, i, 0)),
        pl.BlockSpec((2, RB, FH), lambda i: (0, i, 0)),
        pl.BlockSpec((RB, 8), lambda i: (i, 0)),
        pl.BlockSpec((1, 256), lambda i: (0, 0)),
    ],
    out_specs=pl.BlockSpec((RB, 256), lambda i: (i, 0)),
    out_shape=jax.ShapeDtypeStruct((NPAD, 256), jnp.float32),
)


def kernel(x, edge_index, W1, b1, W2, b2, W3, b3, W4, b4, W5, b5, W6, b6):
    x_p = jnp.zeros((NPAD, 128), jnp.float32).at[:N].set(x)
    src = edge_index[0]
    dst = edge_index[1]
    npad_e = ROWS2D * CHUNK - E
    pidx = jnp.arange(npad_e, dtype=jnp.int32)
    src2d = jnp.concatenate(
        [src, pidx % N]).reshape(ROWS2D, CHUNK)
    dst2d = jnp.concatenate(
        [dst, N + pidx % (NPAD - N)]).reshape(ROWS2D, CHUNK)
    ones_blk = jnp.ones((CHUNK, DW), jnp.float32)
    zeros64 = jnp.zeros((64, FH), jnp.float32)

    deg = _sc_deg(dst2d, ones_blk, zeros64)
    g, dinv8 = _tc1(x_p, W1, deg)
    for wn, bn in ((W2, b1), (W3, b2), (W4, b3), (W5, b4), (W6, b5)):
        sst = _sc_agg(g, src2d, dst2d, zeros64)
        g = _tcmid(sst, g, dinv8, bn.reshape(1, 256), wn)
    sst = _sc_agg(g, src2d, dst2d, zeros64)
    out = _tcfin(sst, g, dinv8, b6.reshape(1, 256))
    return out[:N]
